# Initial kernel scaffold; baseline (speedup 1.0000x reference)
import jax
import jax.numpy as jnp
from jax.experimental import pallas as pl


def kernel(x, adjacency, weight, gate_weight):
    raise NotImplementedError("write your pallas kernel here")


# speedup vs baseline: 1.8985x; 1.8985x over previous
import functools

import jax
import jax.numpy as jnp
from jax.experimental import pallas as pl


def _proj_kernel(x_ref, w_ref, wg_ref, sg_ref, *, d_out):
    xb = x_ref[...]
    s = jnp.dot(xb, w_ref[...], preferred_element_type=jnp.float32)
    g = jnp.dot(xb, wg_ref[...], preferred_element_type=jnp.float32)
    sg_ref[...] = jnp.concatenate([s, g], axis=1).astype(jnp.bfloat16)


def _spmm_gate_kernel(adj_ref, sg_ref, out_ref, *, d_out):
    adj = adj_ref[...].astype(jnp.bfloat16)
    acc = jnp.dot(adj, sg_ref[...], preferred_element_type=jnp.float32)
    out_ref[...] = acc[:, :d_out] * jax.nn.sigmoid(acc[:, d_out:])


def kernel(x, adjacency, weight, gate_weight):
    n, d_in = x.shape
    d_out = weight.shape[1]

    bp = min(2000, n)
    sg = pl.pallas_call(
        functools.partial(_proj_kernel, d_out=d_out),
        grid=(n // bp,),
        in_specs=[
            pl.BlockSpec((bp, d_in), lambda i: (i, 0)),
            pl.BlockSpec((d_in, d_out), lambda i: (0, 0)),
            pl.BlockSpec((d_in, d_out), lambda i: (0, 0)),
        ],
        out_specs=pl.BlockSpec((bp, 2 * d_out), lambda i: (i, 0)),
        out_shape=jax.ShapeDtypeStruct((n, 2 * d_out), jnp.bfloat16),
    )(x, weight, gate_weight)

    bm = min(200, n)
    out = pl.pallas_call(
        functools.partial(_spmm_gate_kernel, d_out=d_out),
        grid=(n // bm,),
        in_specs=[
            pl.BlockSpec((bm, n), lambda i: (i, 0)),
            pl.BlockSpec((n, 2 * d_out), lambda i: (0, 0)),
        ],
        out_specs=pl.BlockSpec((bm, d_out), lambda i: (i, 0)),
        out_shape=jax.ShapeDtypeStruct((n, d_out), jnp.float32),
    )(adjacency, sg)
    return out


# speedup vs baseline: 1.9938x; 1.0502x over previous
import functools

import jax
import jax.numpy as jnp
from jax.experimental import pallas as pl
from jax.experimental.pallas import tpu as pltpu


def _gated_spmm_kernel(x_ref, w_ref, wg_ref, adj_ref, out_ref, sg_ref, *, d_out):
    @pl.when(pl.program_id(0) == 0)
    def _():
        xb = x_ref[...]
        s = jnp.dot(xb, w_ref[...], preferred_element_type=jnp.float32)
        g = jnp.dot(xb, wg_ref[...], preferred_element_type=jnp.float32)
        sg_ref[...] = jnp.concatenate([s, g], axis=1).astype(jnp.bfloat16)

    adj = adj_ref[...].astype(jnp.bfloat16)
    acc = jnp.dot(adj, sg_ref[...], preferred_element_type=jnp.float32)
    out_ref[...] = acc[:, :d_out] * jax.nn.sigmoid(acc[:, d_out:])


def kernel(x, adjacency, weight, gate_weight):
    n, d_in = x.shape
    d_out = weight.shape[1]

    bm = min(200, n)
    out = pl.pallas_call(
        functools.partial(_gated_spmm_kernel, d_out=d_out),
        grid=(n // bm,),
        in_specs=[
            pl.BlockSpec((n, d_in), lambda i: (0, 0)),
            pl.BlockSpec((d_in, d_out), lambda i: (0, 0)),
            pl.BlockSpec((d_in, d_out), lambda i: (0, 0)),
            pl.BlockSpec((bm, n), lambda i: (i, 0)),
        ],
        out_specs=pl.BlockSpec((bm, d_out), lambda i: (i, 0)),
        out_shape=jax.ShapeDtypeStruct((n, d_out), jnp.float32),
        scratch_shapes=[pltpu.VMEM((n, 2 * d_out), jnp.bfloat16)],
    )(x, weight, gate_weight, adjacency)
    return out


# speedup vs baseline: 2.0158x; 1.0110x over previous
import functools

import jax
import jax.numpy as jnp
from jax.experimental import pallas as pl
from jax.experimental.pallas import tpu as pltpu


def _gated_spmm_kernel(x_ref, w_ref, wg_ref, adj_ref, out_ref, sg_ref, *, d_out):
    @pl.when(pl.program_id(0) == 0)
    def _():
        xb = x_ref[...]
        s = jnp.dot(xb, w_ref[...], preferred_element_type=jnp.float32)
        g = jnp.dot(xb, wg_ref[...], preferred_element_type=jnp.float32)
        sg_ref[...] = jnp.concatenate([s, g], axis=1).astype(jnp.bfloat16)

    adj = adj_ref[...].astype(jnp.bfloat16)
    acc = jnp.dot(adj, sg_ref[...], preferred_element_type=jnp.float32)
    out_ref[...] = acc[:, :d_out] * jax.nn.sigmoid(acc[:, d_out:])


def kernel(x, adjacency, weight, gate_weight):
    n, d_in = x.shape
    d_out = weight.shape[1]

    bm = min(400, n)
    out = pl.pallas_call(
        functools.partial(_gated_spmm_kernel, d_out=d_out),
        grid=(n // bm,),
        in_specs=[
            pl.BlockSpec((n, d_in), lambda i: (0, 0)),
            pl.BlockSpec((d_in, d_out), lambda i: (0, 0)),
            pl.BlockSpec((d_in, d_out), lambda i: (0, 0)),
            pl.BlockSpec((bm, n), lambda i: (i, 0)),
        ],
        out_specs=pl.BlockSpec((bm, d_out), lambda i: (i, 0)),
        out_shape=jax.ShapeDtypeStruct((n, d_out), jnp.float32),
        scratch_shapes=[pltpu.VMEM((n, 2 * d_out), jnp.bfloat16)],
    )(x, weight, gate_weight, adjacency)
    return out
